# Initial kernel scaffold; baseline (speedup 1.0000x reference)
import jax
import jax.numpy as jnp
from jax.experimental import pallas as pl


def kernel(pred0, pred1, pred2, gt_boxes, gt_cls):
    raise NotImplementedError("write your pallas kernel here")


# speedup vs baseline: 9.1315x; 9.1315x over previous
import functools

import jax
import jax.numpy as jnp
from jax import lax
from jax.experimental import pallas as pl
from jax.experimental.pallas import tpu as pltpu
from jax.experimental.pallas import tpu_sc as plsc

_IMG = 416.0
_ANCH = [(14.0, 18.0), (34.0, 41.0), (53.0, 88.0), (92.0, 56.0),
         (104.0, 125.0), (126.0, 226.0), (234.0, 151.0), (216.0, 298.0),
         (375.0, 362.0)]
_MASKS = [(6, 7, 8), (3, 4, 5), (0, 1, 2)]
_WS = (13, 26, 52)
_B, _N = 8, 20
_LC, _LN = 5.0, 0.5

_NW = 32
_BN = _B * _N
_ROWS = 1024
_PT = _ROWS // _NW


def _sp(x):
    return jnp.maximum(x, 0.0) + jnp.log1p(jnp.exp(-jnp.abs(x)))


def _bce(x, z):
    return jnp.maximum(x, 0.0) - x * z + jnp.log1p(jnp.exp(-jnp.abs(x)))


def _assign_body(p0, p1, p2, gxr, gyr, gwr, ghr, gclsr,
                 conf_o, t0_o, t1_o, t2_o, iw0_o, iig_o, islf_o,
                 win_o, cwin_o, iwin_o, bsel_o, cslf_o,
                 tx_o, ty_o, tw_o, th_o):
    h = pl.program_id(0)
    v0 = p0[...]
    v1 = p1[...]
    v2 = p2[...]
    t0_o[:, :, :, 0:255] = v0
    t1_o[:, :, :, 0:255] = v1
    t2_o[:, :, :, 0:255] = v2

    c12 = jnp.float32(0.0)
    c0 = jnp.float32(0.0)
    for aa in range(3):
        ch = 85 * aa + 4
        c12 = c12 + jnp.sum(_sp(v1[..., ch])) + jnp.sum(_sp(v2[..., ch]))
        c0 = c0 + jnp.sum(_sp(v0[..., ch]))

    @pl.when(h == 0)
    def _():
        conf_o[...] = jnp.zeros((1, 1), jnp.float32)

    conf_o[...] = conf_o[...] + jnp.full((1, 1), _LN * c12, jnp.float32)

    @pl.when(h < 13)
    def _():
        conf_o[...] = conf_o[...] + jnp.full((1, 1), _LN * c0, jnp.float32)

    @pl.when(h == 0)
    def _():
        gx = gxr[...]
        gy = gyr[...]
        gw = gwr[...]
        gh = ghr[...]
        gcls = gclsr[...]
        bidx = lax.broadcasted_iota(jnp.int32, (_B, _N), 0)
        ltr = (lax.broadcasted_iota(jnp.int32, (_B, _N, _N), 1)
               < lax.broadcasted_iota(jnp.int32, (_B, _N, _N), 2))
        clseq = gcls[:, :, None] == gcls[:, None, :]
        for s in range(3):
            W = _WS[s]
            awl = [_ANCH[m][0] / _IMG for m in _MASKS[s]]
            ahl = [_ANCH[m][1] / _IMG for m in _MASKS[s]]
            ious = []
            for aa in range(3):
                inter = jnp.minimum(gw, awl[aa]) * jnp.minimum(gh, ahl[aa])
                union = gw * gh + awl[aa] * ahl[aa] - inter + 1e-16
                ious.append(inter / union)
            best = jnp.zeros((_B, _N), jnp.int32)
            bv = ious[0]
            for aa in (1, 2):
                better = ious[aa] > bv
                best = jnp.where(better, aa, best)
                bv = jnp.maximum(bv, ious[aa])
            gi = jnp.minimum((gx * W).astype(jnp.int32), W - 1)
            gj = jnp.minimum((gy * W).astype(jnp.int32), W - 1)
            cell = gj * W + gi
            key = best * (W * W) + cell
            same_key = key[:, :, None] == key[:, None, :]
            win = ~jnp.any(same_key & ltr, axis=-1)
            cwin = ~jnp.any(same_key & clseq & ltr, axis=-1)
            same_cell = cell[:, :, None] == cell[:, None, :]
            aw_b = jnp.where(best == 1, awl[1], awl[0])
            aw_b = jnp.where(best == 2, awl[2], aw_b)
            ah_b = jnp.where(best == 1, ahl[1], ahl[0])
            ah_b = jnp.where(best == 2, ahl[2], ah_b)
            tx_o[s] = gx * W - gi.astype(jnp.float32)
            ty_o[s] = gy * W - gj.astype(jnp.float32)
            tw_o[s] = jnp.log(gw / aw_b)
            th_o[s] = jnp.log(gh / ah_b)
            win_o[s] = win.astype(jnp.float32)
            cwin_o[s] = cwin.astype(jnp.float32)
            bsel_o[s] = best
            rbase = cell * 16 + bidx
            iw0_o[s] = rbase
            cself = 85 * best + 5 + gcls
            islf_o[s] = rbase + 8 * (cself // 128)
            cslf_o[s] = cself % 128
            for aa in range(3):
                ig = ious[aa] > 0.5
                lat = jnp.any(same_cell & ltr & ig[:, None, :], axis=-1)
                inobj = jnp.any(same_cell & (best[:, None, :] == aa), axis=-1)
                iwin_o[s, aa] = (ig & ~lat & ~inobj).astype(jnp.float32)
                iig_o[s, aa] = rbase + (8 if aa == 2 else 0)


def _assign(pt0, pt1, pt2, gx, gy, gw, gh, gcls):
    f32 = jnp.float32
    i32 = jnp.int32
    out_shape = [
        jax.ShapeDtypeStruct((1, 1), f32),
        jax.ShapeDtypeStruct((13, 13, _B, 256), f32),
        jax.ShapeDtypeStruct((26, 26, _B, 256), f32),
        jax.ShapeDtypeStruct((52, 52, _B, 256), f32),
        jax.ShapeDtypeStruct((3, _B, _N), i32),
        jax.ShapeDtypeStruct((3, 3, _B, _N), i32),
        jax.ShapeDtypeStruct((3, _B, _N), i32),
        jax.ShapeDtypeStruct((3, _B, _N), f32),
        jax.ShapeDtypeStruct((3, _B, _N), f32),
        jax.ShapeDtypeStruct((3, 3, _B, _N), f32),
        jax.ShapeDtypeStruct((3, _B, _N), i32),
        jax.ShapeDtypeStruct((3, _B, _N), i32),
        jax.ShapeDtypeStruct((3, _B, _N), f32),
        jax.ShapeDtypeStruct((3, _B, _N), f32),
        jax.ShapeDtypeStruct((3, _B, _N), f32),
        jax.ShapeDtypeStruct((3, _B, _N), f32),
    ]
    s0_map = lambda h: (jnp.minimum(h, 12), 0, 0, 0)
    row_map = lambda h: (h, 0, 0, 0)
    gt_map = lambda h: (0, 0)
    in_specs = [
        pl.BlockSpec((1, 13, _B, 255), s0_map),
        pl.BlockSpec((1, 26, _B, 255), row_map),
        pl.BlockSpec((2, 52, _B, 255), row_map),
        pl.BlockSpec((_B, _N), gt_map),
        pl.BlockSpec((_B, _N), gt_map),
        pl.BlockSpec((_B, _N), gt_map),
        pl.BlockSpec((_B, _N), gt_map),
        pl.BlockSpec((_B, _N), gt_map),
    ]
    out_specs = [
        pl.BlockSpec((1, 1), lambda h: (0, 0)),
        pl.BlockSpec((1, 13, _B, 256), s0_map),
        pl.BlockSpec((1, 26, _B, 256), row_map),
        pl.BlockSpec((2, 52, _B, 256), row_map),
    ] + [
        pl.BlockSpec(o.shape, lambda h, nd=len(o.shape): (0,) * nd)
        for o in out_shape[4:]
    ]
    return pl.pallas_call(
        _assign_body,
        grid=(26,),
        in_specs=in_specs,
        out_specs=out_specs,
        out_shape=out_shape,
    )(pt0, pt1, pt2, gx, gy, gw, gh, gcls)


def _gather_sc(t0v, t1v, t2v, idx):
    mesh = plsc.VectorSubcoreMesh(core_axis_name="c", subcore_axis_name="s")

    @functools.partial(
        pl.kernel,
        out_type=jax.ShapeDtypeStruct((3 * _ROWS, 128), jnp.float32),
        mesh=mesh,
        scratch_types=[
            pltpu.VMEM((_PT,), jnp.int32),
            pltpu.VMEM((_PT, 128), jnp.float32),
            pltpu.SemaphoreType.DMA,
        ],
    )
    def k2(t0_h, t1_h, t2_h, idx_h, out_h, idx_v, rows_v, sem):
        wid = lax.axis_index("s") * 2 + lax.axis_index("c")
        for s, p in enumerate((t0_h, t1_h, t2_h)):
            off = s * _ROWS + wid * _PT
            pltpu.sync_copy(idx_h.at[pl.ds(off, _PT)], idx_v)
            pltpu.async_copy(p.at[idx_v], rows_v, sem).wait()
            pltpu.sync_copy(rows_v, out_h.at[pl.ds(off, _PT)])

    return k2(t0v, t1v, t2v, idx)


def _combine_body(gw0r, gw1r, gigr, gslfr, winr, cwinr, iwinr, bselr, cslfr,
                  txr, tyr, twr, thr, cbr, out):
    w0 = gw0r[...]
    w1 = gw1r[...]
    best = bselr[...]
    win = winr[...]
    ga0 = w0[..., 0:85]
    ga1 = jnp.concatenate([w0[..., 85:128], w1[..., 0:42]], axis=-1)
    ga2 = w1[..., 42:127]
    b1 = best == 1
    b2 = best == 2
    g85 = jnp.where(b2, ga2, jnp.where(b1, ga1, ga0))
    px = g85[..., 0]
    py = g85[..., 1]
    pw = g85[..., 2]
    ph = g85[..., 3]
    pc = g85[..., 4]
    clssum = jnp.sum(_sp(g85[..., 5:]), axis=-1)
    posval = (_LC * (_bce(px, txr[...]) + _bce(py, tyr[...])
                     + (pw - twr[...]) ** 2 + (ph - thr[...]) ** 2)
              + _sp(-pc) - _LN * _sp(pc) + clssum)
    gig = gigr[...]
    iwin = iwinr[...]
    isub = (jnp.sum(iwin[:, 0] * _sp(gig[:, 0, ..., 4]))
            + jnp.sum(iwin[:, 1] * _sp(gig[:, 1, ..., 89]))
            + jnp.sum(iwin[:, 2] * _sp(gig[:, 2, ..., 46])))
    lanes = lax.broadcasted_iota(jnp.int32, (3, _B, _N, 128), 3)
    onehot = (lanes == cslfr[...]).astype(jnp.float32)
    pcslf = jnp.sum(gslfr[...] * onehot, axis=-1)
    total = (jnp.sum(cbr[...])
             + jnp.sum(win * posval)
             - jnp.sum(cwinr[...] * pcslf)
             - _LN * isub)
    out[...] = jnp.full((1, 1), total / jnp.sum(win), jnp.float32)


def _combine(gw0, gw1, gig, gslf, win, cwin, iwin, bsel, cslf,
             tx, ty, tw, th, conf_b):
    return pl.pallas_call(
        _combine_body,
        out_shape=jax.ShapeDtypeStruct((1, 1), jnp.float32),
    )(gw0, gw1, gig, gslf, win, cwin, iwin, bsel, cslf, tx, ty, tw, th, conf_b)


def _tile_view(t, hh, ww):
    return (t.reshape(hh, ww, _B, 2, 128).transpose(0, 1, 3, 2, 4)
            .reshape(hh * ww * 16, 128))


def kernel(pred0, pred1, pred2, gt_boxes, gt_cls):
    gx = gt_boxes[..., 0]
    gy = gt_boxes[..., 1]
    gw = gt_boxes[..., 2]
    gh = gt_boxes[..., 3]
    gcls = gt_cls.astype(jnp.int32)
    pt0 = pred0.transpose(2, 3, 0, 1)
    pt1 = pred1.transpose(2, 3, 0, 1)
    pt2 = pred2.transpose(2, 3, 0, 1)
    (conf_b, t0, t1, t2, iw0, iig, islf, win, cwin, iwin, bsel, cslf,
     tx, ty, tw, th) = _assign(pt0, pt1, pt2, gx, gy, gw, gh, gcls)
    idx = jnp.concatenate(
        [iw0.reshape(3, _BN), iw0.reshape(3, _BN) + 8, iig.reshape(3, 3 * _BN),
         islf.reshape(3, _BN), jnp.zeros((3, _ROWS - 6 * _BN), jnp.int32)],
        axis=1).reshape(-1)
    g = _gather_sc(_tile_view(t0, 13, 13), _tile_view(t1, 26, 26),
                   _tile_view(t2, 52, 52), idx).reshape(3, _ROWS, 128)
    gw0 = g[:, 0:_BN].reshape(3, _B, _N, 128)
    gw1 = g[:, _BN:2 * _BN].reshape(3, _B, _N, 128)
    gig = g[:, 2 * _BN:5 * _BN].reshape(3, 3, _B, _N, 128)
    gslf = g[:, 5 * _BN:6 * _BN].reshape(3, _B, _N, 128)
    out = _combine(gw0, gw1, gig, gslf, win, cwin, iwin,
                   bsel.reshape(3, _B, _N, 1), cslf.reshape(3, _B, _N, 1),
                   tx, ty, tw, th, conf_b)
    return out[0, 0]
